# baseline (speedup 1.0000x reference)
import functools

import jax
import jax.numpy as jnp
from jax import lax
from jax.experimental import pallas as pl
from jax.experimental.pallas import tpu as pltpu
from jax.experimental.pallas import tpu_sc as plsc

_NC = 2
_NS = 16
_NW = _NC * _NS
_CR = 32


def _sc_body(s_len, pos_hbm, kv_hbm, vv_hbm, kc_hbm, vc_hbm, ok_hbm, ov_hbm,
             buf0, buf1, pos_v, ls0, ls1, ss0, ss1):
    wid = lax.axis_index("s") * _NC + lax.axis_index("c")
    rows_total = kc_hbm.shape[0]
    rows_w = rows_total // _NW
    per_batch = s_len // rows_w
    base = wid * rows_w
    b = wid // per_batch
    l0 = (wid % per_batch) * rows_w
    q = kv_hbm.shape[1]
    nchunks = rows_w // _CR
    bufs = (buf0, buf1)
    lsems = (ls0, ls1)
    ssems = (ss0, ss1)

    pltpu.sync_copy(pos_hbm, pos_v)
    pos_vec = pos_v[...]
    lane = lax.iota(jnp.int32, 16)
    ps = [jnp.max(jnp.where(lane == i, pos_vec, jnp.int32(-1))) for i in range(q)]

    for src, dst in ((kc_hbm, ok_hbm), (vc_hbm, ov_hbm)):

        def group(g, _, src=src, dst=dst):
            for u in range(2):
                off = base + (g * 2 + u) * _CR
                pltpu.make_async_copy(src.at[pl.ds(off, _CR)], bufs[u], lsems[u]).start()
            for u in range(2):
                off = base + (g * 2 + u) * _CR
                pltpu.make_async_copy(src.at[pl.ds(off, _CR)], bufs[u], lsems[u]).wait()
                pltpu.make_async_copy(bufs[u], dst.at[pl.ds(off, _CR)], ssems[u]).start()
            for u in range(2):
                off = base + (g * 2 + u) * _CR
                pltpu.make_async_copy(bufs[u], dst.at[pl.ds(off, _CR)], ssems[u]).wait()
            return 0

        lax.fori_loop(0, nchunks // 2, group, 0)

    for val, dst in ((kv_hbm, ok_hbm), (vv_hbm, ov_hbm)):
        for i in range(q):
            p = ps[i]

            @pl.when((p >= l0) & (p < l0 + rows_w))
            def _(val=val, dst=dst, i=i, p=p):
                pltpu.sync_copy(
                    val.at[b, pl.ds(i, 1)],
                    dst.at[pl.ds(base + (p - l0), 1)],
                )


def kernel(input_pos, k_val, v_val, k_cache, v_cache):
    B, S, H, D = k_cache.shape
    Q = k_val.shape[1]
    F = H * D
    kc = k_cache.reshape(B * S, F)
    vc = v_cache.reshape(B * S, F)
    kv = k_val.reshape(B, Q, F)
    vv = v_val.reshape(B, Q, F)
    out_k, out_v = pl.kernel(
        functools.partial(_sc_body, S),
        out_type=[
            jax.ShapeDtypeStruct((B * S, F), jnp.float32),
            jax.ShapeDtypeStruct((B * S, F), jnp.float32),
        ],
        mesh=plsc.VectorSubcoreMesh(core_axis_name="c", subcore_axis_name="s"),
        compiler_params=pltpu.CompilerParams(needs_layout_passes=False),
        scratch_types=[
            pltpu.VMEM((_CR, F), jnp.float32),
            pltpu.VMEM((_CR, F), jnp.float32),
            pltpu.VMEM((16,), jnp.int32),
            pltpu.SemaphoreType.DMA,
            pltpu.SemaphoreType.DMA,
            pltpu.SemaphoreType.DMA,
            pltpu.SemaphoreType.DMA,
        ],
    )(input_pos, kv, vv, kc, vc)
    return (out_k.reshape(B, S, H, D), out_v.reshape(B, S, H, D))
